# baseline (speedup 1.0000x reference)
import functools

import jax
import jax.numpy as jnp
from jax import lax
from jax.experimental import pallas as pl
from jax.experimental.pallas import tpu as pltpu
from jax.experimental.pallas import tpu_sc as plsc

NC = 2
NS = 16
L = 16
NW = NC * NS
B = 16384
D = 64
BPW = B // NW
CHUNK = 128
NCHUNK = BPW // CHUNK
NBLK = BPW // L


def _score_body(users_hbm, items_hbm, ub_hbm, ib_hbm, ue_hbm, ie_hbm,
                out_hbm, uidx, iidx, u_rows, i_rows, ub, ib, out_v, tsc,
                sem):
    wid = lax.axis_index("s") * NC + lax.axis_index("c")
    base = wid * BPW

    pltpu.sync_copy(users_hbm.at[pl.ds(wid * NCHUNK, NCHUNK)], uidx)
    pltpu.sync_copy(items_hbm.at[pl.ds(wid * NCHUNK, NCHUNK)], iidx)

    copies = []
    for k in range(NCHUNK):
        copies.append(pltpu.async_copy(
            ue_hbm.at[uidx.at[k]], u_rows.at[pl.ds(k * CHUNK, CHUNK)], sem))
        copies.append(pltpu.async_copy(
            ie_hbm.at[iidx.at[k]], i_rows.at[pl.ds(k * CHUNK, CHUNK)], sem))
        copies.append(pltpu.async_copy(
            ub_hbm.at[uidx.at[k]], ub.at[pl.ds(k * CHUNK, CHUNK)], sem))
        copies.append(pltpu.async_copy(
            ib_hbm.at[iidx.at[k]], ib.at[pl.ds(k * CHUNK, CHUNK)], sem))
    for c in copies:
        c.wait()

    rows17 = lax.iota(jnp.int32, L) * (L + 1)

    def block(b, carry):
        e0 = pl.multiple_of(b * L, L)
        acc = ub[pl.ds(e0, L)] + ib[pl.ds(e0, L)]
        for e in range(L):
            er = e0 + e
            s = u_rows[er, pl.ds(0, L)] * i_rows[er, pl.ds(0, L)]
            for q in range(1, D // L):
                s = s + u_rows[er, pl.ds(q * L, L)] * i_rows[er, pl.ds(q * L, L)]
            tsc[pl.ds(e * (L + 1), L)] = s
        for j in range(L):
            col = plsc.load_gather(tsc, [rows17 + j])
            acc = acc + col
        out_v[pl.ds(e0, L)] = acc
        return carry

    lax.fori_loop(0, NBLK, block, 0)
    pltpu.sync_copy(out_v, out_hbm.at[pl.ds(base, BPW)])


_score_kernel = functools.partial(
    pl.kernel,
    out_type=jax.ShapeDtypeStruct((B,), jnp.float32),
    mesh=plsc.VectorSubcoreMesh(core_axis_name="c", subcore_axis_name="s"),
    compiler_params=pltpu.CompilerParams(
        needs_layout_passes=False, use_tc_tiling_on_sc=False),
    scratch_types=[
        pltpu.VMEM((NCHUNK, CHUNK), jnp.int32),
        pltpu.VMEM((NCHUNK, CHUNK), jnp.int32),
        pltpu.VMEM((BPW, D), jnp.float32),
        pltpu.VMEM((BPW, D), jnp.float32),
        pltpu.VMEM((BPW,), jnp.float32),
        pltpu.VMEM((BPW,), jnp.float32),
        pltpu.VMEM((BPW,), jnp.float32),
        pltpu.VMEM((L * (L + 1),), jnp.float32),
        pltpu.SemaphoreType.DMA,
    ],
)(_score_body)


def kernel(users, items, u_bias_w, i_bias_w, u_embed_w, i_embed_w):
    users2d = users.astype(jnp.int32).reshape(NW * NCHUNK, CHUNK)
    items2d = items.astype(jnp.int32).reshape(NW * NCHUNK, CHUNK)
    return _score_kernel(
        users2d, items2d,
        u_bias_w.T.reshape(-1), i_bias_w.T.reshape(-1),
        u_embed_w, i_embed_w)


# speedup vs baseline: 1.2290x; 1.2290x over previous
import functools

import jax
import jax.numpy as jnp
from jax import lax
from jax.experimental import pallas as pl
from jax.experimental.pallas import tpu as pltpu
from jax.experimental.pallas import tpu_sc as plsc

NC = 2
NS = 16
L = 16
NW = NC * NS
B = 16384
D = 64
BPW = B // NW
CHUNK = 128
NCH = BPW // CHUNK
BLKS = CHUNK // L
NCHUNK = BPW // CHUNK
RING = 256


def _score_body(users_hbm, items_hbm, ue_hbm, ie_hbm, ub_hbm, ib_hbm,
                out_hbm, uidx, iidx, ua, ia, ubuf, ibuf, out_v, tsc, dume,
                sem, sem2, sem3, semp3, semp4):
    wid = lax.axis_index("s") * NC + lax.axis_index("c")
    base = wid * BPW
    pltpu.sync_copy(users_hbm.at[pl.ds(wid * NCHUNK, NCHUNK)], uidx)
    pltpu.sync_copy(items_hbm.at[pl.ds(wid * NCHUNK, NCHUNK)], iidx)

    rows17 = lax.iota(jnp.int32, L) * (L + 1)

    bias_copies = []
    for k in range(NCHUNK):
        bias_copies.append(pltpu.async_copy(
            ub_hbm.at[uidx.at[k]], ubuf.at[pl.ds(k * CHUNK, CHUNK)], sem3))
        bias_copies.append(pltpu.async_copy(
            ib_hbm.at[iidx.at[k]], ibuf.at[pl.ds(k * CHUNK, CHUNK)], sem3))

    NBLK = BPW // L

    def fire(b, s):
        uvec = uidx[b // BLKS, pl.ds((b % BLKS) * L, L)]
        ivec = iidx[b // BLKS, pl.ds((b % BLKS) * L, L)]
        s0 = (b % (RING // L)) * L
        for e in range(L):
            u = uvec[e]
            pltpu.async_copy(ue_hbm.at[u], ua.at[s0 + e], s)
            it = ivec[e]
            pltpu.async_copy(ie_hbm.at[it], ia.at[s0 + e], s)

    def drain(s):
        pltpu.make_async_copy(ue_hbm.at[pl.ds(0, 2 * L)], dume, s).wait()

    def compute(b):
        e0 = pl.multiple_of((b % (RING // L)) * L, L)
        for e in range(L):
            s = ua[e0 + e, pl.ds(0, L)] * ia[e0 + e, pl.ds(0, L)]
            for q in range(1, D // L):
                s = s + (ua[e0 + e, pl.ds(q * L, L)]
                         * ia[e0 + e, pl.ds(q * L, L)])
            tsc[pl.ds(e * (L + 1), L)] = s
        acc = plsc.load_gather(tsc, [rows17])
        for jj in range(1, L):
            acc = acc + plsc.load_gather(tsc, [rows17 + jj])
        out_v[pl.ds(b * L, L)] = acc

    sems = (sem, sem2, semp3, semp4)
    fire(0, sems[0])
    fire(1, sems[1])

    def step(j, carry):
        b = 4 * j
        for t in range(4):
            fire(b + 2 + t, sems[(2 + t) % 4])
            drain(sems[t % 4])
            compute(b + t)
        return carry

    lax.fori_loop(0, NBLK // 4 - 1, step, 0)
    b0 = NBLK - 4
    fire(NBLK - 2, sems[2])
    drain(sems[0])
    compute(b0)
    fire(NBLK - 1, sems[3])
    drain(sems[1])
    compute(b0 + 1)
    drain(sems[2])
    compute(b0 + 2)
    drain(sems[3])
    compute(b0 + 3)

    for cp in bias_copies:
        cp.wait()

    def addb(b, carry):
        e0 = pl.multiple_of(b * L, L)
        out_v[pl.ds(e0, L)] = (out_v[pl.ds(e0, L)] + ubuf[pl.ds(e0, L)]
                               + ibuf[pl.ds(e0, L)])
        return carry

    lax.fori_loop(0, BPW // L, addb, 0)
    pltpu.sync_copy(out_v, out_hbm.at[pl.ds(base, BPW)])


def _make_kernel():
    mesh = plsc.VectorSubcoreMesh(core_axis_name="c", subcore_axis_name="s")
    return functools.partial(
        pl.kernel,
        out_type=jax.ShapeDtypeStruct((B,), jnp.float32),
        mesh=mesh,
        compiler_params=pltpu.CompilerParams(needs_layout_passes=False),
        scratch_types=[
            pltpu.VMEM((NCHUNK, CHUNK), jnp.int32),
            pltpu.VMEM((NCHUNK, CHUNK), jnp.int32),
            pltpu.VMEM((RING, D), jnp.float32),
            pltpu.VMEM((RING, D), jnp.float32),
            pltpu.VMEM((BPW,), jnp.float32),
            pltpu.VMEM((BPW,), jnp.float32),
            pltpu.VMEM((BPW,), jnp.float32),
            pltpu.VMEM((L * (L + 1),), jnp.float32),
            pltpu.VMEM((2 * L, D), jnp.float32),
            pltpu.SemaphoreType.DMA,
            pltpu.SemaphoreType.DMA,
            pltpu.SemaphoreType.DMA,
            pltpu.SemaphoreType.DMA,
            pltpu.SemaphoreType.DMA,
        ],
    )(_score_body)


_score_kernel = None


def kernel(users, items, u_bias_w, i_bias_w, u_embed_w, i_embed_w):
    global _score_kernel
    if _score_kernel is None:
        _score_kernel = _make_kernel()
    users2d = users.astype(jnp.int32).reshape(NW * NCHUNK, CHUNK)
    items2d = items.astype(jnp.int32).reshape(NW * NCHUNK, CHUNK)
    ub_flat = u_bias_w.T.reshape(-1)
    ib_flat = i_bias_w.T.reshape(-1)
    return _score_kernel(users2d, items2d, u_embed_w, i_embed_w,
                         ub_flat, ib_flat)
